# Initial kernel scaffold; baseline (speedup 1.0000x reference)
import jax
import jax.numpy as jnp
from jax.experimental import pallas as pl


def kernel(x, edge_attr, params, edge_index):
    raise NotImplementedError("write your pallas kernel here")


# speedup vs baseline: 3.1664x; 3.1664x over previous
import functools
import math

import jax
import jax.numpy as jnp
from jax import lax
from jax.experimental import pallas as pl
from jax.experimental.pallas import tpu as pltpu
from jax.experimental.pallas import tpu_sc as plsc

N, E, DIN, DE, H, K, Z = 10000, 320000, 128, 16, 128, 10, 32

BN = 1000
BE = 2000

_NCORES = 2
_NSUB = 16
_NW = _NCORES * _NSUB
_CHUNK = 80
_GROUP = 5
_GC = _CHUNK * _GROUP
_NP = 10240

def _sc_mesh():
    return plsc.VectorSubcoreMesh(core_axis_name="c", subcore_axis_name="s")


def _ln(h, g, b):
    mu = jnp.mean(h, axis=-1, keepdims=True)
    var = jnp.mean((h - mu) ** 2, axis=-1, keepdims=True)
    return (h - mu) * lax.rsqrt(var + 1e-5) * g + b


def _mlp_ln_body(x_ref, w1_ref, b1_ref, w2_ref, b2_ref, g_ref, bb_ref, o_ref):
    h = jnp.maximum(jnp.dot(x_ref[...], w1_ref[...],
                            preferred_element_type=jnp.float32) + b1_ref[...], 0.0)
    h = jnp.dot(h, w2_ref[...], preferred_element_type=jnp.float32) + b2_ref[...]
    o_ref[...] = _ln(h, g_ref[...], bb_ref[...])


def _mlp_ln(xa, p, bm):
    r, din = xa.shape
    dh = p["W1"].shape[1]
    dout = p["W2"].shape[1]
    full = lambda shape: pl.BlockSpec(shape, lambda i: (0, 0))
    return pl.pallas_call(
        _mlp_ln_body,
        grid=(r // bm,),
        in_specs=[
            pl.BlockSpec((bm, din), lambda i: (i, 0)),
            full((din, dh)), full((1, dh)), full((dh, dout)), full((1, dout)),
            full((1, dout)), full((1, dout)),
        ],
        out_specs=pl.BlockSpec((bm, dout), lambda i: (i, 0)),
        out_shape=jax.ShapeDtypeStruct((r, dout), jnp.float32),
    )(xa, p["W1"], p["b1"].reshape(1, dh), p["W2"], p["b2"].reshape(1, dout),
      p["g"].reshape(1, dout), p["b"].reshape(1, dout))


def _proj_body(h_ref, ws_ref, wd_ref, ps_ref, pd_ref):
    h = h_ref[...]
    ps_ref[...] = jnp.dot(h, ws_ref[...], preferred_element_type=jnp.float32)
    pd_ref[...] = jnp.dot(h, wd_ref[...], preferred_element_type=jnp.float32)


def _proj(h, ws, wd):
    full = lambda shape: pl.BlockSpec(shape, lambda i: (0, 0))
    out = jax.ShapeDtypeStruct((N, H), jnp.float32)
    return pl.pallas_call(
        _proj_body,
        grid=(N // BN,),
        in_specs=[pl.BlockSpec((BN, H), lambda i: (i, 0)), full((H, H)), full((H, H))],
        out_specs=[pl.BlockSpec((BN, H), lambda i: (i, 0))] * 2,
        out_shape=[out, out],
    )(h, ws, wd)


def _edge_body(gs_ref, gd_ref, e_ref, we_ref, b1_ref, w2_ref, b2_ref, g_ref,
               bb_ref, en_ref, eo_ref):
    e = e_ref[...]
    t = gs_ref[...] + gd_ref[...] + jnp.dot(
        e, we_ref[...], preferred_element_type=jnp.float32) + b1_ref[...]
    u = jnp.dot(jnp.maximum(t, 0.0), w2_ref[...],
                preferred_element_type=jnp.float32) + b2_ref[...]
    en = _ln(u, g_ref[...], bb_ref[...])
    en_ref[...] = en
    if eo_ref is not None:
        eo_ref[...] = e + en


def _edge_update(gs, gd, e, we, p, residual):
    full = lambda shape: pl.BlockSpec(shape, lambda i: (0, 0))
    row = pl.BlockSpec((BE, H), lambda i: (i, 0))
    out = jax.ShapeDtypeStruct((E, H), jnp.float32)
    body = _edge_body if residual else functools.partial(_edge_body, eo_ref=None)
    return pl.pallas_call(
        body,
        grid=(E // BE,),
        in_specs=[row, row, row, full((H, H)), full((1, H)), full((H, H)),
                  full((1, H)), full((1, H)), full((1, H))],
        out_specs=[row, row] if residual else [row],
        out_shape=[out, out] if residual else [out],
    )(gs, gd, e, we, p["b1"].reshape(1, H), p["W2"], p["b2"].reshape(1, H),
      p["g"].reshape(1, H), p["b"].reshape(1, H))


def _node_body(h_ref, a0_ref, a1_ref, wh_ref, wa_ref, b1_ref, w2_ref, b2_ref,
               g_ref, bb_ref, o_ref):
    h = h_ref[...]
    agg = a0_ref[...] + a1_ref[...]
    t = (jnp.dot(h, wh_ref[...], preferred_element_type=jnp.float32)
         + jnp.dot(agg, wa_ref[...], preferred_element_type=jnp.float32)
         + b1_ref[...])
    u = jnp.dot(jnp.maximum(t, 0.0), w2_ref[...],
                preferred_element_type=jnp.float32) + b2_ref[...]
    o_ref[...] = h + _ln(u, g_ref[...], bb_ref[...])


def _node_update(h, a0, a1, p):
    full = lambda shape: pl.BlockSpec(shape, lambda i: (0, 0))
    row = pl.BlockSpec((BN, H), lambda i: (i, 0))
    wh, wa = p["W1"][:H], p["W1"][H:]
    return pl.pallas_call(
        _node_body,
        grid=(N // BN,),
        in_specs=[row, row, row, full((H, H)), full((H, H)), full((1, H)),
                  full((H, H)), full((1, H)), full((1, H)), full((1, H))],
        out_specs=row,
        out_shape=jax.ShapeDtypeStruct((N, H), jnp.float32),
    )(h, a0, a1, wh, wa, p["b1"].reshape(1, H), p["W2"], p["b2"].reshape(1, H),
      p["g"].reshape(1, H), p["b"].reshape(1, H))


_DOUT = K * (1 + 2 * Z)


def _pool_body(h_ref, gw_ref, w1_ref, b1_ref, w2_ref, b2_ref, mask_ref, o_ref,
               m_ref, s_ref, p_ref):
    i = pl.program_id(0)

    @pl.when(i == 0)
    def _():
        m_ref[0] = -1e30
        s_ref[0] = 0.0
        p_ref[...] = jnp.zeros_like(p_ref)

    h = h_ref[...]
    sloc = jnp.sum(h * gw_ref[...], axis=-1, keepdims=True)
    m_old = m_ref[0]
    m_new = jnp.maximum(m_old, jnp.max(sloc))
    c = jnp.exp(m_old - m_new)
    w = jnp.exp(sloc - m_new)
    s_ref[0] = s_ref[0] * c + jnp.sum(w)
    p_ref[...] = p_ref[...] * c + jnp.sum(w * h, axis=0, keepdims=True)
    m_ref[0] = m_new

    @pl.when(i == pl.num_programs(0) - 1)
    def _():
        pooled = p_ref[...] / s_ref[0]
        hh = jnp.maximum(jnp.dot(pooled, w1_ref[...],
                                 preferred_element_type=jnp.float32) + b1_ref[...], 0.0)
        raw = jnp.dot(hh, w2_ref[...],
                      preferred_element_type=jnp.float32) + b2_ref[...]
        o_ref[...] = jnp.where(mask_ref[...] > 0.0,
                               jnp.clip(raw, math.log(0.05), 5.0), raw)


def _pool_head(h, gw_row, p, mask):
    full = lambda shape: pl.BlockSpec(shape, lambda i: (0, 0))
    return pl.pallas_call(
        _pool_body,
        grid=(N // BN,),
        in_specs=[pl.BlockSpec((BN, H), lambda i: (i, 0)), full((1, H)),
                  full((H, H)), full((1, H)), full((H, _DOUT)), full((1, _DOUT)),
                  full((1, _DOUT))],
        out_specs=full((1, _DOUT)),
        out_shape=jax.ShapeDtypeStruct((1, _DOUT), jnp.float32),
        scratch_shapes=[pltpu.SMEM((1,), jnp.float32),
                        pltpu.SMEM((1,), jnp.float32),
                        pltpu.VMEM((1, H), jnp.float32)],
    )(h, gw_row, p["W1"], p["b1"].reshape(1, H), p["W2"],
      p["b2"].reshape(1, _DOUT), mask)


def _sc_gather_pair(ps, pd, src, dst):
    out = jax.ShapeDtypeStruct((E, H), jnp.float32)
    rows_per_w = E // _NW

    @functools.partial(
        pl.kernel,
        out_type=[out, out],
        mesh=_sc_mesh(),
        scratch_types=[
            pltpu.VMEM((_GC,), jnp.int32),
            pltpu.VMEM((_GC,), jnp.int32),
            pltpu.VMEM((_GC, H), jnp.float32),
            pltpu.VMEM((_GC, H), jnp.float32),
            pltpu.SemaphoreType.DMA,
            pltpu.SemaphoreType.DMA,
        ],
    )
    def k(ps_hbm, pd_hbm, src_hbm, dst_hbm, gs_hbm, gd_hbm,
          is_v, id_v, a_v, b_v, s1, s2):
        wid = lax.axis_index("s") * _NCORES + lax.axis_index("c")

        @pl.loop(0, rows_per_w // _GC)
        def _(kk):
            off = wid * rows_per_w + kk * _GC
            pltpu.sync_copy(src_hbm.at[pl.ds(off, _GC)], is_v)
            pltpu.sync_copy(dst_hbm.at[pl.ds(off, _GC)], id_v)
            copies = []
            for j in range(_GROUP):
                sl = pl.ds(j * _CHUNK, _CHUNK)
                copies.append(pltpu.async_copy(ps_hbm.at[is_v.at[sl]], a_v.at[sl], s1))
                copies.append(pltpu.async_copy(pd_hbm.at[id_v.at[sl]], b_v.at[sl], s2))
            for cp in copies:
                cp.wait()
            pltpu.sync_copy(a_v, gs_hbm.at[pl.ds(off, _GC)])
            pltpu.sync_copy(b_v, gd_hbm.at[pl.ds(off, _GC)])

    return k(ps, pd, src, dst)


def _sc_segsum(en, dst, zrows):
    rows_per_w = E // _NW
    np_ = _NP
    zn = np_ // _NSUB

    @functools.partial(
        pl.kernel,
        out_type=jax.ShapeDtypeStruct((_NCORES, np_, H), jnp.float32),
        mesh=_sc_mesh(),
        scratch_types=[
            pltpu.VMEM((_CHUNK,), jnp.int32),
            pltpu.VMEM((_CHUNK, H), jnp.float32),
            pltpu.VMEM_SHARED((np_, H), jnp.float32),
            pltpu.SemaphoreType.DMA,
        ],
    )
    def k(en_hbm, dst_hbm, z_hbm, out_hbm, idx_v, buf_v, acc_sh, sem):
        cid = lax.axis_index("c")
        sid = lax.axis_index("s")
        pltpu.sync_copy(z_hbm, acc_sh.at[pl.ds(sid * zn, zn)])
        plsc.subcore_barrier()
        base = cid * (E // _NCORES) + sid * rows_per_w

        @pl.loop(0, rows_per_w // _CHUNK)
        def _(kk):
            off = base + kk * _CHUNK
            pltpu.sync_copy(dst_hbm.at[pl.ds(off, _CHUNK)], idx_v)
            pltpu.sync_copy(en_hbm.at[pl.ds(off, _CHUNK)], buf_v)
            pltpu.sync_copy(buf_v, acc_sh.at[idx_v], add=True)

        plsc.subcore_barrier()
        pltpu.sync_copy(acc_sh.at[pl.ds(sid * zn, zn)],
                        out_hbm.at[cid].at[pl.ds(sid * zn, zn)])

    return k(en, dst, zrows)


def kernel(x, edge_attr, params, edge_index):
    src = edge_index[0]
    dst = edge_index[1]

    h = _mlp_ln(x, params["ne"], BN)
    e = _mlp_ln(edge_attr, params["ee"], BE)

    zrows = jnp.zeros((_NP // _NSUB, H), jnp.float32)
    for bi, blk in enumerate(params["mp"]):
        w1 = blk["edge"]["W1"]
        ps, pd = _proj(h, w1[:H], w1[H:2 * H])
        gs, gd = _sc_gather_pair(ps, pd, src, dst)
        if bi < 2:
            en, e = _edge_update(gs, gd, e, w1[2 * H:], blk["edge"], residual=True)
        else:
            (en,) = _edge_update(gs, gd, e, w1[2 * H:], blk["edge"], residual=False)
        parts = _sc_segsum(en, dst, zrows)
        h = _node_update(h, parts[0, :N], parts[1, :N], blk["node"])

    mask = (jnp.arange(_DOUT) % (1 + 2 * Z) >= 1 + Z).astype(jnp.float32).reshape(1, _DOUT)
    raw = _pool_head(h, params["gate_W"].reshape(1, H), params["head"], mask)
    raw = raw.reshape(1, K, 1 + 2 * Z)
    return raw[:, :, 0], raw[:, :, 1:1 + Z], raw[:, :, 1 + Z:]


# speedup vs baseline: 3.2088x; 1.0134x over previous
import functools
import math

import jax
import jax.numpy as jnp
from jax import lax
from jax.experimental import pallas as pl
from jax.experimental.pallas import tpu as pltpu
from jax.experimental.pallas import tpu_sc as plsc

N, E, DIN, DE, H, K, Z = 10000, 320000, 128, 16, 128, 10, 32

BN = 1000
BE = 2000

_NCORES = 2
_NSUB = 16
_NW = _NCORES * _NSUB
_CHUNK = 80
_GROUP = 5
_GC = _CHUNK * _GROUP
_NP = 10240
_GCH = 40
_GSET = 200

def _sc_mesh():
    return plsc.VectorSubcoreMesh(core_axis_name="c", subcore_axis_name="s")


def _ln(h, g, b):
    mu = jnp.mean(h, axis=-1, keepdims=True)
    var = jnp.mean((h - mu) ** 2, axis=-1, keepdims=True)
    return (h - mu) * lax.rsqrt(var + 1e-5) * g + b


def _mlp_ln_body(x_ref, w1_ref, b1_ref, w2_ref, b2_ref, g_ref, bb_ref, o_ref):
    h = jnp.maximum(jnp.dot(x_ref[...], w1_ref[...],
                            preferred_element_type=jnp.float32) + b1_ref[...], 0.0)
    h = jnp.dot(h, w2_ref[...], preferred_element_type=jnp.float32) + b2_ref[...]
    o_ref[...] = _ln(h, g_ref[...], bb_ref[...])


def _mlp_ln(xa, p, bm):
    r, din = xa.shape
    dh = p["W1"].shape[1]
    dout = p["W2"].shape[1]
    full = lambda shape: pl.BlockSpec(shape, lambda i: (0, 0))
    return pl.pallas_call(
        _mlp_ln_body,
        grid=(r // bm,),
        in_specs=[
            pl.BlockSpec((bm, din), lambda i: (i, 0)),
            full((din, dh)), full((1, dh)), full((dh, dout)), full((1, dout)),
            full((1, dout)), full((1, dout)),
        ],
        out_specs=pl.BlockSpec((bm, dout), lambda i: (i, 0)),
        out_shape=jax.ShapeDtypeStruct((r, dout), jnp.float32),
    )(xa, p["W1"], p["b1"].reshape(1, dh), p["W2"], p["b2"].reshape(1, dout),
      p["g"].reshape(1, dout), p["b"].reshape(1, dout))


def _proj_body(h_ref, ws_ref, wd_ref, ps_ref, pd_ref):
    h = h_ref[...]
    ps_ref[...] = jnp.dot(h, ws_ref[...], preferred_element_type=jnp.float32)
    pd_ref[...] = jnp.dot(h, wd_ref[...], preferred_element_type=jnp.float32)


def _proj(h, ws, wd):
    full = lambda shape: pl.BlockSpec(shape, lambda i: (0, 0))
    out = jax.ShapeDtypeStruct((N, H), jnp.float32)
    return pl.pallas_call(
        _proj_body,
        grid=(N // BN,),
        in_specs=[pl.BlockSpec((BN, H), lambda i: (i, 0)), full((H, H)), full((H, H))],
        out_specs=[pl.BlockSpec((BN, H), lambda i: (i, 0))] * 2,
        out_shape=[out, out],
    )(h, ws, wd)


def _edge_body(gs_ref, gd_ref, e_ref, we_ref, b1_ref, w2_ref, b2_ref, g_ref,
               bb_ref, en_ref, eo_ref):
    e = e_ref[...]
    t = gs_ref[...] + gd_ref[...] + jnp.dot(
        e, we_ref[...], preferred_element_type=jnp.float32) + b1_ref[...]
    u = jnp.dot(jnp.maximum(t, 0.0), w2_ref[...],
                preferred_element_type=jnp.float32) + b2_ref[...]
    en = _ln(u, g_ref[...], bb_ref[...])
    en_ref[...] = en
    if eo_ref is not None:
        eo_ref[...] = e + en


def _edge_update(gs, gd, e, we, p, residual):
    full = lambda shape: pl.BlockSpec(shape, lambda i: (0, 0))
    row = pl.BlockSpec((BE, H), lambda i: (i, 0))
    out = jax.ShapeDtypeStruct((E, H), jnp.float32)
    body = _edge_body if residual else functools.partial(_edge_body, eo_ref=None)
    return pl.pallas_call(
        body,
        grid=(E // BE,),
        in_specs=[row, row, row, full((H, H)), full((1, H)), full((H, H)),
                  full((1, H)), full((1, H)), full((1, H))],
        out_specs=[row, row] if residual else [row],
        out_shape=[out, out] if residual else [out],
    )(gs, gd, e, we, p["b1"].reshape(1, H), p["W2"], p["b2"].reshape(1, H),
      p["g"].reshape(1, H), p["b"].reshape(1, H))


def _node_body(h_ref, a0_ref, a1_ref, wh_ref, wa_ref, b1_ref, w2_ref, b2_ref,
               g_ref, bb_ref, o_ref):
    h = h_ref[...]
    agg = a0_ref[...] + a1_ref[...]
    t = (jnp.dot(h, wh_ref[...], preferred_element_type=jnp.float32)
         + jnp.dot(agg, wa_ref[...], preferred_element_type=jnp.float32)
         + b1_ref[...])
    u = jnp.dot(jnp.maximum(t, 0.0), w2_ref[...],
                preferred_element_type=jnp.float32) + b2_ref[...]
    o_ref[...] = h + _ln(u, g_ref[...], bb_ref[...])


def _node_update(h, a0, a1, p):
    full = lambda shape: pl.BlockSpec(shape, lambda i: (0, 0))
    row = pl.BlockSpec((BN, H), lambda i: (i, 0))
    wh, wa = p["W1"][:H], p["W1"][H:]
    return pl.pallas_call(
        _node_body,
        grid=(N // BN,),
        in_specs=[row, row, row, full((H, H)), full((H, H)), full((1, H)),
                  full((H, H)), full((1, H)), full((1, H)), full((1, H))],
        out_specs=row,
        out_shape=jax.ShapeDtypeStruct((N, H), jnp.float32),
    )(h, a0, a1, wh, wa, p["b1"].reshape(1, H), p["W2"], p["b2"].reshape(1, H),
      p["g"].reshape(1, H), p["b"].reshape(1, H))


_DOUT = K * (1 + 2 * Z)


def _pool_body(h_ref, gw_ref, w1_ref, b1_ref, w2_ref, b2_ref, mask_ref, o_ref,
               m_ref, s_ref, p_ref):
    i = pl.program_id(0)

    @pl.when(i == 0)
    def _():
        m_ref[0] = -1e30
        s_ref[0] = 0.0
        p_ref[...] = jnp.zeros_like(p_ref)

    h = h_ref[...]
    sloc = jnp.sum(h * gw_ref[...], axis=-1, keepdims=True)
    m_old = m_ref[0]
    m_new = jnp.maximum(m_old, jnp.max(sloc))
    c = jnp.exp(m_old - m_new)
    w = jnp.exp(sloc - m_new)
    s_ref[0] = s_ref[0] * c + jnp.sum(w)
    p_ref[...] = p_ref[...] * c + jnp.sum(w * h, axis=0, keepdims=True)
    m_ref[0] = m_new

    @pl.when(i == pl.num_programs(0) - 1)
    def _():
        pooled = p_ref[...] / s_ref[0]
        hh = jnp.maximum(jnp.dot(pooled, w1_ref[...],
                                 preferred_element_type=jnp.float32) + b1_ref[...], 0.0)
        raw = jnp.dot(hh, w2_ref[...],
                      preferred_element_type=jnp.float32) + b2_ref[...]
        o_ref[...] = jnp.where(mask_ref[...] > 0.0,
                               jnp.clip(raw, math.log(0.05), 5.0), raw)


def _pool_head(h, gw_row, p, mask):
    full = lambda shape: pl.BlockSpec(shape, lambda i: (0, 0))
    return pl.pallas_call(
        _pool_body,
        grid=(N // BN,),
        in_specs=[pl.BlockSpec((BN, H), lambda i: (i, 0)), full((1, H)),
                  full((H, H)), full((1, H)), full((H, _DOUT)), full((1, _DOUT)),
                  full((1, _DOUT))],
        out_specs=full((1, _DOUT)),
        out_shape=jax.ShapeDtypeStruct((1, _DOUT), jnp.float32),
        scratch_shapes=[pltpu.SMEM((1,), jnp.float32),
                        pltpu.SMEM((1,), jnp.float32),
                        pltpu.VMEM((1, H), jnp.float32)],
    )(h, gw_row, p["W1"], p["b1"].reshape(1, H), p["W2"],
      p["b2"].reshape(1, _DOUT), mask)


def _sc_gather_pair(ps, pd, src, dst):
    out = jax.ShapeDtypeStruct((E, H), jnp.float32)
    rows_per_w = E // _NW

    npairs = rows_per_w // (2 * _GSET)

    @functools.partial(
        pl.kernel,
        out_type=[out, out],
        mesh=_sc_mesh(),
        scratch_types=[
            pltpu.VMEM((_GSET,), jnp.int32),
            pltpu.VMEM((_GSET,), jnp.int32),
            pltpu.VMEM((_GSET,), jnp.int32),
            pltpu.VMEM((_GSET,), jnp.int32),
            pltpu.VMEM((_GSET, H), jnp.float32),
            pltpu.VMEM((_GSET, H), jnp.float32),
            pltpu.VMEM((_GSET, H), jnp.float32),
            pltpu.VMEM((_GSET, H), jnp.float32),
            pltpu.SemaphoreType.DMA,
            pltpu.SemaphoreType.DMA,
            pltpu.SemaphoreType.DMA,
            pltpu.SemaphoreType.DMA,
        ],
    )
    def k(ps_hbm, pd_hbm, src_hbm, dst_hbm, gs_hbm, gd_hbm,
          is0, id0, is1, id1, a0, b0, a1, b1, g0, g1, w0, w1):
        wid = lax.axis_index("s") * _NCORES + lax.axis_index("c")
        base = wid * rows_per_w
        sets = ((is0, id0, a0, b0, g0, w0), (is1, id1, a1, b1, g1, w1))

        def fire(off, s):
            isv, idv, a_v, b_v, gsem, _ = sets[s]
            pltpu.sync_copy(src_hbm.at[pl.ds(off, _GSET)], isv)
            pltpu.sync_copy(dst_hbm.at[pl.ds(off, _GSET)], idv)
            hs = []
            for j in range(_GSET // _GCH):
                sl = pl.ds(j * _GCH, _GCH)
                hs.append(pltpu.async_copy(ps_hbm.at[isv.at[sl]], a_v.at[sl], gsem))
                hs.append(pltpu.async_copy(pd_hbm.at[idv.at[sl]], b_v.at[sl], gsem))
            return hs

        def writeback(off, s, wait_handles):
            _, _, a_v, b_v, _, wsem = sets[s]
            for h in wait_handles:
                h.wait()
            pltpu.async_copy(a_v, gs_hbm.at[pl.ds(off, _GSET)], wsem)
            pltpu.async_copy(b_v, gd_hbm.at[pl.ds(off, _GSET)], wsem)

        def drain_wb(s):
            _, _, a_v, b_v, _, wsem = sets[s]
            pltpu.make_async_copy(a_v, gs_hbm.at[pl.ds(base, _GSET)], wsem).wait()
            pltpu.make_async_copy(b_v, gd_hbm.at[pl.ds(base, _GSET)], wsem).wait()

        h0 = fire(base, 0)
        h1 = fire(base + _GSET, 1)
        writeback(base, 0, h0)
        writeback(base + _GSET, 1, h1)

        @pl.loop(1, npairs)
        def _(jj):
            off = base + jj * (2 * _GSET)
            drain_wb(0)
            h0 = fire(off, 0)
            drain_wb(1)
            h1 = fire(off + _GSET, 1)
            writeback(off, 0, h0)
            writeback(off + _GSET, 1, h1)

        drain_wb(0)
        drain_wb(1)

    return k(ps, pd, src, dst)


def _sc_segsum(en, dst, zrows):
    rows_per_w = E // _NW
    np_ = _NP
    zn = np_ // _NSUB

    @functools.partial(
        pl.kernel,
        out_type=jax.ShapeDtypeStruct((_NCORES, np_, H), jnp.float32),
        mesh=_sc_mesh(),
        scratch_types=[
            pltpu.VMEM((_CHUNK,), jnp.int32),
            pltpu.VMEM((_CHUNK, H), jnp.float32),
            pltpu.VMEM_SHARED((np_, H), jnp.float32),
            pltpu.SemaphoreType.DMA,
        ],
    )
    def k(en_hbm, dst_hbm, z_hbm, out_hbm, idx_v, buf_v, acc_sh, sem):
        cid = lax.axis_index("c")
        sid = lax.axis_index("s")
        pltpu.sync_copy(z_hbm, acc_sh.at[pl.ds(sid * zn, zn)])
        plsc.subcore_barrier()
        base = cid * (E // _NCORES) + sid * rows_per_w

        @pl.loop(0, rows_per_w // _CHUNK)
        def _(kk):
            off = base + kk * _CHUNK
            pltpu.sync_copy(dst_hbm.at[pl.ds(off, _CHUNK)], idx_v)
            pltpu.sync_copy(en_hbm.at[pl.ds(off, _CHUNK)], buf_v)
            pltpu.sync_copy(buf_v, acc_sh.at[idx_v], add=True)

        plsc.subcore_barrier()
        pltpu.sync_copy(acc_sh.at[pl.ds(sid * zn, zn)],
                        out_hbm.at[cid].at[pl.ds(sid * zn, zn)])

    return k(en, dst, zrows)


def kernel(x, edge_attr, params, edge_index):
    src = edge_index[0]
    dst = edge_index[1]

    h = _mlp_ln(x, params["ne"], BN)
    e = _mlp_ln(edge_attr, params["ee"], BE)

    zrows = jnp.zeros((_NP // _NSUB, H), jnp.float32)
    for bi, blk in enumerate(params["mp"]):
        w1 = blk["edge"]["W1"]
        ps, pd = _proj(h, w1[:H], w1[H:2 * H])
        gs, gd = _sc_gather_pair(ps, pd, src, dst)
        if bi < 2:
            en, e = _edge_update(gs, gd, e, w1[2 * H:], blk["edge"], residual=True)
        else:
            (en,) = _edge_update(gs, gd, e, w1[2 * H:], blk["edge"], residual=False)
        parts = _sc_segsum(en, dst, zrows)
        h = _node_update(h, parts[0, :N], parts[1, :N], blk["node"])

    mask = (jnp.arange(_DOUT) % (1 + 2 * Z) >= 1 + Z).astype(jnp.float32).reshape(1, _DOUT)
    raw = _pool_head(h, params["gate_W"].reshape(1, H), params["head"], mask)
    raw = raw.reshape(1, K, 1 + 2 * Z)
    return raw[:, :, 0], raw[:, :, 1:1 + Z], raw[:, :, 1 + Z:]
